# baseline (speedup 1.0000x reference)
import jax
import jax.numpy as jnp
from jax import lax
from jax.experimental import pallas as pl
from jax.experimental.pallas import tpu as pltpu
from jax.experimental.pallas import tpu_sc as plsc

N = 10000
NV = N // 16
TOPK = 32
KOUT = 17
NC, NS = 2, 16
NW = NC * NS
BLK_OUT = 8 * KOUT
UPD = 8
CAPA = 5120
CAPBUF = CAPA + 192

_NEG = float("-inf")


def _sortd(k, i):
    return plsc.sort_key_val(k, i, descending=True)


def _sorta(k, i):
    return plsc.sort_key_val(k, i, descending=False)


def _halver(a_k, a_i, b_k, b_i):
    m = a_k >= b_k
    d_k = jnp.where(m, a_k, b_k)
    d_i = jnp.where(m, a_i, b_i)
    e_k = jnp.where(m, b_k, a_k)
    e_i = jnp.where(m, b_i, a_i)
    return d_k, d_i, e_k, e_i


def _topk_tree(cval, cidx, tval, tidx, off):
    neg16 = jnp.full((16,), _NEG, jnp.float32)

    npad = jnp.maximum(0, (512 - off + 15) // 16)

    def pad(i, c):
        cval[pl.ds(off + i * 16, 16)] = neg16
        return c

    lax.fori_loop(0, npad, pad, jnp.int32(0))

    for p in range(16):
        o = 32 * p
        a_k, a_i = _sortd(cval[pl.ds(o, 16)], cidx[pl.ds(o, 16)])
        b_k, b_i = _sorta(cval[pl.ds(o + 16, 16)], cidx[pl.ds(o + 16, 16)])
        d_k, d_i, e_k, e_i = _halver(a_k, a_i, b_k, b_i)
        hi_k, hi_i = _sortd(d_k, d_i)
        lo_k, lo_i = _sortd(e_k, e_i)
        tval[pl.ds(o, 16)] = hi_k
        tidx[pl.ds(o, 16)] = hi_i
        tval[pl.ds(o + 16, 16)] = lo_k
        tidx[pl.ds(o + 16, 16)] = lo_i

    def merge32(v0, d):
        a_hi_k = tval[pl.ds(16 * v0, 16)]
        a_hi_i = tidx[pl.ds(16 * v0, 16)]
        a_lo_k = tval[pl.ds(16 * v0 + 16, 16)]
        a_lo_i = tidx[pl.ds(16 * v0 + 16, 16)]
        b_hi_k = tval[pl.ds(16 * (v0 + d), 16)]
        b_hi_i = tidx[pl.ds(16 * (v0 + d), 16)]
        b_lo_k = tval[pl.ds(16 * (v0 + d) + 16, 16)]
        b_lo_i = tidx[pl.ds(16 * (v0 + d) + 16, 16)]
        u_k, u_i, _, _ = _halver(a_hi_k, a_hi_i,
                                 lax.rev(b_lo_k, (0,)), lax.rev(b_lo_i, (0,)))
        w_k, w_i, _, _ = _halver(a_lo_k, a_lo_i,
                                 lax.rev(b_hi_k, (0,)), lax.rev(b_hi_i, (0,)))
        su_k, su_i = _sortd(u_k, u_i)
        sw_k, sw_i = _sorta(w_k, w_i)
        d_k, d_i, e_k, e_i = _halver(su_k, su_i, sw_k, sw_i)
        hi_k, hi_i = _sortd(d_k, d_i)
        lo_k, lo_i = _sortd(e_k, e_i)
        return hi_k, hi_i, lo_k, lo_i

    for d in (2, 4, 8):
        for v0 in range(0, 32, 2 * d):
            hi_k, hi_i, lo_k, lo_i = merge32(v0, d)
            tval[pl.ds(16 * v0, 16)] = hi_k
            tidx[pl.ds(16 * v0, 16)] = hi_i
            tval[pl.ds(16 * v0 + 16, 16)] = lo_k
            tidx[pl.ds(16 * v0 + 16, 16)] = lo_i
    hi_k, hi_i, lo_k, lo_i = merge32(0, 16)

    nvx = jnp.maximum(0, (off - 512 + 15) // 16)

    def xbody(s, carry):
        hk, hv, lk, lv = carry
        vk = cval[pl.ds(512 + s * 16, 16)]
        vi = cidx[pl.ds(512 + s * 16, 16)]
        vasc_k, vasc_i = _sorta(vk, vi)
        c_k, c_i, _, _ = _halver(lk, lv, vasc_k, vasc_i)
        ca_k, ca_i = _sorta(c_k, c_i)
        d_k, d_i, e_k, e_i = _halver(hk, hv, ca_k, ca_i)
        hk, hv = _sortd(d_k, d_i)
        lk, lv = _sortd(e_k, e_i)
        return hk, hv, lk, lv

    return lax.fori_loop(0, nvx, xbody, (hi_k, hi_i, lo_k, lo_i))


_IMAX = 2147483647


def _topk32_exact(cval, cidx, off):
    nv = (off + 15) // 16
    iota = lax.iota(jnp.int32, 16)
    neg16 = jnp.full((16,), _NEG, jnp.float32)
    imax16 = jnp.full((16,), _IMAX, jnp.int32)

    def ext(n, carry):
        hi_k, hi_i, lo_k, lo_i = carry

        def p1(s, c):
            bv, bi = c
            v = cval[pl.ds(s * 16, 16)]
            iv = cidx[pl.ds(s * 16, 16)]
            better = (v > bv) | ((v == bv) & (iv < bi))
            return jnp.where(better, v, bv), jnp.where(better, iv, bi)

        bv, bi = lax.fori_loop(0, nv, p1, (neg16, imax16))
        vb = lax.broadcast_in_dim(jnp.max(bv), (16,), ())
        ii = jnp.where(bv == vb, bi, imax16)
        ib = lax.broadcast_in_dim(jnp.min(ii), (16,), ())

        def p2(s, c):
            v = cval[pl.ds(s * 16, 16)]
            iv = cidx[pl.ds(s * 16, 16)]
            hit = (v == vb) & (iv == ib)
            cval[pl.ds(s * 16, 16)] = jnp.where(hit, neg16, v)
            return c

        lax.fori_loop(0, nv, p2, jnp.int32(0))

        lane = iota == (n % 16)

        def upd_hi(hk, hv, lk, lv):
            return jnp.where(lane, vb, hk), jnp.where(lane, ib, hv), lk, lv

        def upd_lo(hk, hv, lk, lv):
            return hk, hv, jnp.where(lane, vb, lk), jnp.where(lane, ib, lv)

        return lax.cond(n < 16, upd_hi, upd_lo, hi_k, hi_i, lo_k, lo_i)

    init = (neg16, jnp.zeros((16,), jnp.int32),
            neg16, jnp.zeros((16,), jnp.int32))
    return lax.fori_loop(0, TOPK, ext, init)


def _body(x_ref, rows_ref, cols_ref, vals_ref,
          rowbuf, cval, cidx, tval, tidx, sbv, sbi, stg_r, stg_c, stg_v, sem):
    wid = lax.axis_index("s") * NC + lax.axis_index("c")
    nblk_w = 40 - (wid >= 2).astype(jnp.int32)
    ns = 8 * nblk_w

    iota = lax.iota(jnp.int32, 16)
    lane0 = iota == 0
    sel16 = jnp.maximum(0, 2 * iota - 1)
    neg16 = jnp.full((16,), _NEG, jnp.float32)

    def row_of(s):
        return 8 * (wid + NW * (s // 8)) + (s % 8)

    def fetch(s, half):
        pltpu.async_copy(
            x_ref.at[pl.ds(row_of(s) * N, N)],
            rowbuf.at[pl.ds(half * N, N)],
            sem,
        )

    def wait_fetch():
        pltpu.make_async_copy(
            x_ref.at[pl.ds(0, N)], rowbuf.at[pl.ds(0, N)], sem
        ).wait()

    fetch(jnp.int32(0), jnp.int32(0))

    def filt8(rbase, base16, m1, m2, tvec, offv, count):
        vs = [rowbuf[pl.ds(rbase + base16 + 16 * u, 16)] for u in range(count)]
        msks = [v >= tvec for v in vs]
        cnts = [plsc.all_reduce_population_count(m) for m in msks]
        advs = [jnp.where(c > 0, 16, 0) for c in cnts]
        for v in vs:
            nm1 = jnp.maximum(m1, v)
            m2 = jnp.maximum(m2, jnp.minimum(m1, v))
            m1 = nm1
        offs = []
        for a in advs:
            offs.append(offv)
            offv = offv + a
        offv = jnp.minimum(offv, CAPA)
        for u in range(count):
            plsc.store_scatter(cval, [offs[u] + iota], vs[u])
            plsc.store_scatter(cidx, [offs[u] + iota], iota + (base16 + 16 * u))
        return m1, m2, offv

    def compact(nvs4, tvec):
        def cbody(it, offm1):
            s = it * 64
            vs = [cval[pl.ds(s + 16 * u, 16)] for u in range(4)]
            ivs = [cidx[pl.ds(s + 16 * u, 16)] for u in range(4)]
            msks = [v >= tvec for v in vs]
            poss = [plsc.cumsum(m.astype(jnp.int32)) for m in msks]
            cnts = [plsc.all_reduce_population_count(m) for m in msks]
            for u in range(4):
                idx = jnp.minimum(offm1 + poss[u], CAPA - 16)
                plsc.store_scatter(cval, [idx], vs[u], mask=msks[u])
                plsc.store_scatter(cidx, [idx], ivs[u], mask=msks[u])
                offm1 = offm1 + cnts[u]
            return offm1

        offm1 = lax.fori_loop(0, nvs4, cbody, jnp.full((16,), -1, jnp.int32))
        return jnp.minimum(jnp.max(offm1) + 1, CAPA - 16)

    def process_row(s):
        half = (s % 2) * N
        def fbody(it, carry):
            m1, m2, tpend, offv = carry
            tvec = tpend
            m1, m2, offv = filt8(half, it * (UPD * 16), m1, m2, tvec, offv, UPD)
            tpend = lax.broadcast_in_dim(jnp.min(m2), (16,), ())
            return m1, m2, tpend, offv

        carry = (neg16, neg16, neg16, jnp.zeros((16,), jnp.int32))
        m1, m2, tvec, offv = lax.fori_loop(0, NV // UPD, fbody, carry)
        m1, m2, offv = filt8(half, (NV - 1) * 16, m1, m2, tvec, offv, 1)
        nvs = jnp.max(offv) // 16
        tfin = lax.broadcast_in_dim(jnp.min(m2), (16,), ())
        for u in range(3):
            cval[pl.ds(nvs * 16 + 16 * u, 16)] = neg16
        off = compact((nvs + 3) // 4, tfin)

        hi_k, hi_i, lo_k, lo_i = _topk_tree(cval, cidx, tval, tidx, off)

        nv = (off + 15) // 16
        v32 = lax.broadcast_in_dim(jnp.min(lo_k), (16,), ())

        def cge(s, c):
            v = cval[pl.ds(s * 16, 16)]
            return c + plsc.all_reduce_population_count(v >= v32)

        cnt_ge = lax.fori_loop(0, nv, cge, jnp.zeros((16,), jnp.int32))
        sbv[pl.ds(0, 16)] = hi_k
        sbv[pl.ds(16, 16)] = lo_k
        e1 = plsc.load_gather(sbv, [iota]) == plsc.load_gather(sbv, [iota + 1])
        e2 = (plsc.load_gather(sbv, [iota + 16])
              == plsc.load_gather(sbv, [jnp.minimum(iota + 17, 31)]))
        eqc = plsc.all_reduce_population_count(e1 | (e2 & (iota < 15)))
        tied = (jnp.max(cnt_ge) != 32) | (jnp.max(eqc) > 0)

        hi_k, hi_i, lo_k, lo_i = lax.cond(
            tied,
            lambda: _topk32_exact(cval, cidx, off),
            lambda: (hi_k, hi_i, lo_k, lo_i),
        )

        sbv[pl.ds(0, 16)] = hi_k
        sbv[pl.ds(16, 16)] = lo_k
        sbi[pl.ds(0, 16)] = hi_i
        sbi[pl.ds(16, 16)] = lo_i
        q = s % 8
        qo = q * KOUT
        stg_v[pl.ds(qo, 16)] = plsc.load_gather(sbv, [sel16])
        stg_c[pl.ds(qo, 16)] = plsc.load_gather(sbi, [sel16])
        pos31 = lax.broadcast_in_dim(jnp.int32(31), (16,), ())
        last_pos = lax.broadcast_in_dim(qo + 16, (16,), ())
        plsc.store_scatter(stg_v, [last_pos], plsc.load_gather(sbv, [pos31]),
                           mask=lane0)
        plsc.store_scatter(stg_c, [last_pos], plsc.load_gather(sbi, [pos31]),
                           mask=lane0)
        row_id = row_of(s)
        stg_r[pl.ds(qo, 16)] = lax.broadcast_in_dim(row_id, (16,), ())
        plsc.store_scatter(stg_r, [last_pos],
                           lax.broadcast_in_dim(row_id, (16,), ()), mask=lane0)

    def sbody(s, carry):
        wait_fetch()

        @pl.when(s + 1 < ns)
        def _():
            fetch(s + 1, (s + 1) % 2)

        process_row(s)

        @pl.when(s % 8 == 7)
        def _():
            b = wid + NW * (s // 8)
            o = b * BLK_OUT
            pltpu.sync_copy(stg_r, rows_ref.at[pl.ds(o, BLK_OUT)])
            pltpu.sync_copy(stg_c, cols_ref.at[pl.ds(o, BLK_OUT)])
            pltpu.sync_copy(stg_v, vals_ref.at[pl.ds(o, BLK_OUT)])

        return carry

    lax.fori_loop(0, ns, sbody, jnp.int32(0))


def kernel(inputs):
    x1d = jnp.reshape(inputs, (-1,))
    mesh = plsc.VectorSubcoreMesh(
        core_axis_name="c", subcore_axis_name="s", num_cores=NC, num_subcores=NS
    )
    kern = pl.kernel(
        _body,
        out_type=(
            jax.ShapeDtypeStruct((N * KOUT,), jnp.int32),
            jax.ShapeDtypeStruct((N * KOUT,), jnp.int32),
            jax.ShapeDtypeStruct((N * KOUT,), jnp.float32),
        ),
        mesh=mesh,
        compiler_params=pltpu.CompilerParams(needs_layout_passes=False),
        scratch_types=[
            pltpu.VMEM((2 * N,), jnp.float32),
            pltpu.VMEM((CAPBUF,), jnp.float32),
            pltpu.VMEM((CAPBUF,), jnp.int32),
            pltpu.VMEM((512,), jnp.float32),
            pltpu.VMEM((512,), jnp.int32),
            pltpu.VMEM((TOPK,), jnp.float32),
            pltpu.VMEM((TOPK,), jnp.int32),
            pltpu.VMEM((BLK_OUT,), jnp.int32),
            pltpu.VMEM((BLK_OUT,), jnp.int32),
            pltpu.VMEM((BLK_OUT,), jnp.float32),
            pltpu.SemaphoreType.DMA,
        ],
    )
    rows, cols, values = kern(x1d)
    return rows, cols, values


# speedup vs baseline: 1.0939x; 1.0939x over previous
import jax
import jax.numpy as jnp
from jax import lax
from jax.experimental import pallas as pl
from jax.experimental.pallas import tpu as pltpu
from jax.experimental.pallas import tpu_sc as plsc

N = 10000
NV = N // 16
TOPK = 32
KOUT = 17
NC, NS = 2, 16
NW = NC * NS
BLK_OUT = 8 * KOUT
UPD = 8
CAPA = 5120
CAPBUF = CAPA + 192

_NEG = float("-inf")


def _sortd(k, i):
    return plsc.sort_key_val(k, i, descending=True)


def _sorta(k, i):
    return plsc.sort_key_val(k, i, descending=False)


def _halver(a_k, a_i, b_k, b_i):
    m = a_k >= b_k
    d_k = jnp.where(m, a_k, b_k)
    d_i = jnp.where(m, a_i, b_i)
    e_k = jnp.where(m, b_k, a_k)
    e_i = jnp.where(m, b_i, a_i)
    return d_k, d_i, e_k, e_i


def _topk32_chain(cval, cidx, off):
    cval[pl.ds(off, 16)] = jnp.full((16,), _NEG, jnp.float32)
    nv = (off + 15) // 16

    def body(s, carry):
        hk, hv, lk, lv = carry
        vk = cval[pl.ds(s * 16, 16)]
        vi = cidx[pl.ds(s * 16, 16)]
        vasc_k, vasc_i = _sorta(vk, vi)
        c_k, c_i, _, _ = _halver(lk, lv, vasc_k, vasc_i)
        ca_k, ca_i = _sorta(c_k, c_i)
        d_k, d_i, e_k, e_i = _halver(hk, hv, ca_k, ca_i)
        hk, hv = _sortd(d_k, d_i)
        lk, lv = _sortd(e_k, e_i)
        return hk, hv, lk, lv

    init = (jnp.full((16,), _NEG, jnp.float32), jnp.zeros((16,), jnp.int32),
            jnp.full((16,), _NEG, jnp.float32), jnp.zeros((16,), jnp.int32))
    return lax.fori_loop(0, nv, body, init)


_IMAX = 2147483647


def _topk32_exact(cval, cidx, off):
    nv = (off + 15) // 16
    iota = lax.iota(jnp.int32, 16)
    neg16 = jnp.full((16,), _NEG, jnp.float32)
    imax16 = jnp.full((16,), _IMAX, jnp.int32)

    def ext(n, carry):
        hi_k, hi_i, lo_k, lo_i = carry

        def p1(s, c):
            bv, bi = c
            v = cval[pl.ds(s * 16, 16)]
            iv = cidx[pl.ds(s * 16, 16)]
            better = (v > bv) | ((v == bv) & (iv < bi))
            return jnp.where(better, v, bv), jnp.where(better, iv, bi)

        bv, bi = lax.fori_loop(0, nv, p1, (neg16, imax16))
        vb = lax.broadcast_in_dim(jnp.max(bv), (16,), ())
        ii = jnp.where(bv == vb, bi, imax16)
        ib = lax.broadcast_in_dim(jnp.min(ii), (16,), ())

        def p2(s, c):
            v = cval[pl.ds(s * 16, 16)]
            iv = cidx[pl.ds(s * 16, 16)]
            hit = (v == vb) & (iv == ib)
            cval[pl.ds(s * 16, 16)] = jnp.where(hit, neg16, v)
            return c

        lax.fori_loop(0, nv, p2, jnp.int32(0))

        lane = iota == (n % 16)

        def upd_hi(hk, hv, lk, lv):
            return jnp.where(lane, vb, hk), jnp.where(lane, ib, hv), lk, lv

        def upd_lo(hk, hv, lk, lv):
            return hk, hv, jnp.where(lane, vb, lk), jnp.where(lane, ib, lv)

        return lax.cond(n < 16, upd_hi, upd_lo, hi_k, hi_i, lo_k, lo_i)

    init = (neg16, jnp.zeros((16,), jnp.int32),
            neg16, jnp.zeros((16,), jnp.int32))
    return lax.fori_loop(0, TOPK, ext, init)


def _body(x_ref, rows_ref, cols_ref, vals_ref,
          rowbuf, cval, cidx, sbv, sbi, stg_r, stg_c, stg_v, sem):
    wid = lax.axis_index("s") * NC + lax.axis_index("c")
    nblk_w = 40 - (wid >= 2).astype(jnp.int32)
    ns = 8 * nblk_w

    iota = lax.iota(jnp.int32, 16)
    lane0 = iota == 0
    sel16 = jnp.maximum(0, 2 * iota - 1)
    neg16 = jnp.full((16,), _NEG, jnp.float32)

    def row_of(s):
        return 8 * (wid + NW * (s // 8)) + (s % 8)

    def fetch(s, half):
        pltpu.async_copy(
            x_ref.at[pl.ds(row_of(s) * N, N)],
            rowbuf.at[pl.ds(half * N, N)],
            sem,
        )

    def wait_fetch():
        pltpu.make_async_copy(
            x_ref.at[pl.ds(0, N)], rowbuf.at[pl.ds(0, N)], sem
        ).wait()

    fetch(jnp.int32(0), jnp.int32(0))

    def filt8(rbase, base16, m1, m2, tvec, offv, count):
        vs = [rowbuf[pl.ds(rbase + base16 + 16 * u, 16)] for u in range(count)]
        msks = [v >= tvec for v in vs]
        cnts = [plsc.all_reduce_population_count(m) for m in msks]
        advs = [jnp.where(c > 0, 16, 0) for c in cnts]
        for v in vs:
            nm1 = jnp.maximum(m1, v)
            m2 = jnp.maximum(m2, jnp.minimum(m1, v))
            m1 = nm1
        offs = []
        for a in advs:
            offs.append(offv)
            offv = offv + a
        offv = jnp.minimum(offv, CAPA)
        for u in range(count):
            plsc.store_scatter(cval, [offs[u] + iota], vs[u])
            plsc.store_scatter(cidx, [offs[u] + iota], iota + (base16 + 16 * u))
        return m1, m2, offv

    def compact(nvs4, tvec):
        def cbody(it, offm1):
            s = it * 64
            vs = [cval[pl.ds(s + 16 * u, 16)] for u in range(4)]
            ivs = [cidx[pl.ds(s + 16 * u, 16)] for u in range(4)]
            msks = [v >= tvec for v in vs]
            poss = [plsc.cumsum(m.astype(jnp.int32)) for m in msks]
            cnts = [plsc.all_reduce_population_count(m) for m in msks]
            for u in range(4):
                idx = jnp.minimum(offm1 + poss[u], CAPA - 16)
                plsc.store_scatter(cval, [idx], vs[u], mask=msks[u])
                plsc.store_scatter(cidx, [idx], ivs[u], mask=msks[u])
                offm1 = offm1 + cnts[u]
            return offm1

        offm1 = lax.fori_loop(0, nvs4, cbody, jnp.full((16,), -1, jnp.int32))
        return jnp.minimum(jnp.max(offm1) + 1, CAPA - 16)

    def process_row(s):
        half = (s % 2) * N
        def fbody(it, carry):
            m1, m2, tpend, offv = carry
            tvec = tpend
            m1, m2, offv = filt8(half, it * (UPD * 16), m1, m2, tvec, offv, UPD)
            tpend = lax.broadcast_in_dim(jnp.min(m2), (16,), ())
            return m1, m2, tpend, offv

        carry = (neg16, neg16, neg16, jnp.zeros((16,), jnp.int32))
        m1, m2, tvec, offv = lax.fori_loop(0, NV // UPD, fbody, carry)
        m1, m2, offv = filt8(half, (NV - 1) * 16, m1, m2, tvec, offv, 1)
        nvs = jnp.max(offv) // 16
        tfin = lax.broadcast_in_dim(jnp.min(m2), (16,), ())
        for u in range(3):
            cval[pl.ds(nvs * 16 + 16 * u, 16)] = neg16
        off = compact((nvs + 3) // 4, tfin)

        hi_k, hi_i, lo_k, lo_i = _topk32_chain(cval, cidx, off)

        nv = (off + 15) // 16
        v32 = lax.broadcast_in_dim(jnp.min(lo_k), (16,), ())

        def cge(s, c):
            v = cval[pl.ds(s * 16, 16)]
            return c + plsc.all_reduce_population_count(v >= v32)

        cnt_ge = lax.fori_loop(0, nv, cge, jnp.zeros((16,), jnp.int32))
        sbv[pl.ds(0, 16)] = hi_k
        sbv[pl.ds(16, 16)] = lo_k
        e1 = plsc.load_gather(sbv, [iota]) == plsc.load_gather(sbv, [iota + 1])
        e2 = (plsc.load_gather(sbv, [iota + 16])
              == plsc.load_gather(sbv, [jnp.minimum(iota + 17, 31)]))
        eqc = plsc.all_reduce_population_count(e1 | (e2 & (iota < 15)))
        tied = (jnp.max(cnt_ge) != 32) | (jnp.max(eqc) > 0)

        hi_k, hi_i, lo_k, lo_i = lax.cond(
            tied,
            lambda: _topk32_exact(cval, cidx, off),
            lambda: (hi_k, hi_i, lo_k, lo_i),
        )

        sbv[pl.ds(0, 16)] = hi_k
        sbv[pl.ds(16, 16)] = lo_k
        sbi[pl.ds(0, 16)] = hi_i
        sbi[pl.ds(16, 16)] = lo_i
        q = s % 8
        qo = q * KOUT
        stg_v[pl.ds(qo, 16)] = plsc.load_gather(sbv, [sel16])
        stg_c[pl.ds(qo, 16)] = plsc.load_gather(sbi, [sel16])
        pos31 = lax.broadcast_in_dim(jnp.int32(31), (16,), ())
        last_pos = lax.broadcast_in_dim(qo + 16, (16,), ())
        plsc.store_scatter(stg_v, [last_pos], plsc.load_gather(sbv, [pos31]),
                           mask=lane0)
        plsc.store_scatter(stg_c, [last_pos], plsc.load_gather(sbi, [pos31]),
                           mask=lane0)
        row_id = row_of(s)
        stg_r[pl.ds(qo, 16)] = lax.broadcast_in_dim(row_id, (16,), ())
        plsc.store_scatter(stg_r, [last_pos],
                           lax.broadcast_in_dim(row_id, (16,), ()), mask=lane0)

    def sbody(s, carry):
        wait_fetch()

        @pl.when(s + 1 < ns)
        def _():
            fetch(s + 1, (s + 1) % 2)

        process_row(s)

        @pl.when(s % 8 == 7)
        def _():
            b = wid + NW * (s // 8)
            o = b * BLK_OUT
            pltpu.sync_copy(stg_r, rows_ref.at[pl.ds(o, BLK_OUT)])
            pltpu.sync_copy(stg_c, cols_ref.at[pl.ds(o, BLK_OUT)])
            pltpu.sync_copy(stg_v, vals_ref.at[pl.ds(o, BLK_OUT)])

        return carry

    lax.fori_loop(0, ns, sbody, jnp.int32(0))


def kernel(inputs):
    x1d = jnp.reshape(inputs, (-1,))
    mesh = plsc.VectorSubcoreMesh(
        core_axis_name="c", subcore_axis_name="s", num_cores=NC, num_subcores=NS
    )
    kern = pl.kernel(
        _body,
        out_type=(
            jax.ShapeDtypeStruct((N * KOUT,), jnp.int32),
            jax.ShapeDtypeStruct((N * KOUT,), jnp.int32),
            jax.ShapeDtypeStruct((N * KOUT,), jnp.float32),
        ),
        mesh=mesh,
        compiler_params=pltpu.CompilerParams(needs_layout_passes=False),
        scratch_types=[
            pltpu.VMEM((2 * N,), jnp.float32),
            pltpu.VMEM((CAPBUF,), jnp.float32),
            pltpu.VMEM((CAPBUF,), jnp.int32),
            pltpu.VMEM((TOPK,), jnp.float32),
            pltpu.VMEM((TOPK,), jnp.int32),
            pltpu.VMEM((BLK_OUT,), jnp.int32),
            pltpu.VMEM((BLK_OUT,), jnp.int32),
            pltpu.VMEM((BLK_OUT,), jnp.float32),
            pltpu.SemaphoreType.DMA,
        ],
    )
    rows, cols, values = kern(x1d)
    return rows, cols, values


# speedup vs baseline: 1.1171x; 1.0212x over previous
import jax
import jax.numpy as jnp
from jax import lax
from jax.experimental import pallas as pl
from jax.experimental.pallas import tpu as pltpu
from jax.experimental.pallas import tpu_sc as plsc

N = 10000
NV = N // 16
TOPK = 32
KOUT = 17
NC, NS = 2, 16
NW = NC * NS
BLK_OUT = 8 * KOUT
UPD = 8
CAPA = 5120
CAPBUF = CAPA + 192

_NEG = float("-inf")


def _sortd(k, i):
    return plsc.sort_key_val(k, i, descending=True)


def _sorta(k, i):
    return plsc.sort_key_val(k, i, descending=False)


def _halver(a_k, a_i, b_k, b_i):
    m = a_k >= b_k
    d_k = jnp.where(m, a_k, b_k)
    d_i = jnp.where(m, a_i, b_i)
    e_k = jnp.where(m, b_k, a_k)
    e_i = jnp.where(m, b_i, a_i)
    return d_k, d_i, e_k, e_i


def _topk32_chain(cval, cidx, off):
    cval[pl.ds(off, 16)] = jnp.full((16,), _NEG, jnp.float32)
    nv = (off + 15) // 16

    def body(s, carry):
        hk, hv, lk, lv = carry
        vk = cval[pl.ds(s * 16, 16)]
        vi = cidx[pl.ds(s * 16, 16)]
        vasc_k, vasc_i = _sorta(vk, vi)
        c_k, c_i, _, _ = _halver(lk, lv, vasc_k, vasc_i)
        ca_k, ca_i = _sorta(c_k, c_i)
        d_k, d_i, e_k, e_i = _halver(hk, hv, ca_k, ca_i)
        hk, hv = _sortd(d_k, d_i)
        lk, lv = _sortd(e_k, e_i)
        return hk, hv, lk, lv

    init = (jnp.full((16,), _NEG, jnp.float32), jnp.zeros((16,), jnp.int32),
            jnp.full((16,), _NEG, jnp.float32), jnp.zeros((16,), jnp.int32))
    return lax.fori_loop(0, nv, body, init)


_IMAX = 2147483647


def _topk32_exact(cval, cidx, off):
    nv = (off + 15) // 16
    iota = lax.iota(jnp.int32, 16)
    neg16 = jnp.full((16,), _NEG, jnp.float32)
    imax16 = jnp.full((16,), _IMAX, jnp.int32)

    def ext(n, carry):
        hi_k, hi_i, lo_k, lo_i = carry

        def p1(s, c):
            bv, bi = c
            v = cval[pl.ds(s * 16, 16)]
            iv = cidx[pl.ds(s * 16, 16)]
            better = (v > bv) | ((v == bv) & (iv < bi))
            return jnp.where(better, v, bv), jnp.where(better, iv, bi)

        bv, bi = lax.fori_loop(0, nv, p1, (neg16, imax16))
        vb = lax.broadcast_in_dim(jnp.max(bv), (16,), ())
        ii = jnp.where(bv == vb, bi, imax16)
        ib = lax.broadcast_in_dim(jnp.min(ii), (16,), ())

        def p2(s, c):
            v = cval[pl.ds(s * 16, 16)]
            iv = cidx[pl.ds(s * 16, 16)]
            hit = (v == vb) & (iv == ib)
            cval[pl.ds(s * 16, 16)] = jnp.where(hit, neg16, v)
            return c

        lax.fori_loop(0, nv, p2, jnp.int32(0))

        lane = iota == (n % 16)

        def upd_hi(hk, hv, lk, lv):
            return jnp.where(lane, vb, hk), jnp.where(lane, ib, hv), lk, lv

        def upd_lo(hk, hv, lk, lv):
            return hk, hv, jnp.where(lane, vb, lk), jnp.where(lane, ib, lv)

        return lax.cond(n < 16, upd_hi, upd_lo, hi_k, hi_i, lo_k, lo_i)

    init = (neg16, jnp.zeros((16,), jnp.int32),
            neg16, jnp.zeros((16,), jnp.int32))
    return lax.fori_loop(0, TOPK, ext, init)


def _body(x_ref, rows_ref, cols_ref, vals_ref,
          rowbuf, cval, cidx, sbv, sbi, stg_r, stg_c, stg_v, sem):
    wid = lax.axis_index("s") * NC + lax.axis_index("c")
    nblk_w = 40 - (wid >= 2).astype(jnp.int32)
    ns = 8 * nblk_w

    iota = lax.iota(jnp.int32, 16)
    lane0 = iota == 0
    sel16 = jnp.maximum(0, 2 * iota - 1)
    neg16 = jnp.full((16,), _NEG, jnp.float32)

    def row_of(s):
        return 8 * (wid + NW * (s // 8)) + (s % 8)

    def fetch(s, half):
        pltpu.async_copy(
            x_ref.at[pl.ds(row_of(s) * N, N)],
            rowbuf.at[pl.ds(half * N, N)],
            sem,
        )

    def wait_fetch():
        pltpu.make_async_copy(
            x_ref.at[pl.ds(0, N)], rowbuf.at[pl.ds(0, N)], sem
        ).wait()

    fetch(jnp.int32(0), jnp.int32(0))
    fetch(jnp.int32(1), jnp.int32(1))

    def filt8(rbase, base16, m1, m2, tvec, offv, count):
        vs = [rowbuf[pl.ds(rbase + base16 + 16 * u, 16)] for u in range(count)]
        msks = [v >= tvec for v in vs]
        cnts = [plsc.all_reduce_population_count(m) for m in msks]
        advs = [jnp.where(c > 0, 16, 0) for c in cnts]
        for v in vs:
            nm1 = jnp.maximum(m1, v)
            m2 = jnp.maximum(m2, jnp.minimum(m1, v))
            m1 = nm1
        offs = []
        for a in advs:
            offs.append(offv)
            offv = offv + a
        offv = jnp.minimum(offv, CAPA)
        for u in range(count):
            plsc.store_scatter(cval, [offs[u] + iota], vs[u])
            plsc.store_scatter(cidx, [offs[u] + iota], iota + (base16 + 16 * u))
        return m1, m2, offv

    def compact(nvs4, tvec):
        def cbody(it, offm1):
            s = it * 64
            vs = [cval[pl.ds(s + 16 * u, 16)] for u in range(4)]
            ivs = [cidx[pl.ds(s + 16 * u, 16)] for u in range(4)]
            msks = [v >= tvec for v in vs]
            poss = [plsc.cumsum(m.astype(jnp.int32)) for m in msks]
            cnts = [plsc.all_reduce_population_count(m) for m in msks]
            for u in range(4):
                idx = jnp.minimum(offm1 + poss[u], CAPA - 16)
                plsc.store_scatter(cval, [idx], vs[u], mask=msks[u])
                plsc.store_scatter(cidx, [idx], ivs[u], mask=msks[u])
                offm1 = offm1 + cnts[u]
            return offm1

        offm1 = lax.fori_loop(0, nvs4, cbody, jnp.full((16,), -1, jnp.int32))
        return jnp.minimum(jnp.max(offm1) + 1, CAPA - 16)

    def process_row(s):
        half = (s % 3) * N
        def fbody(it, carry):
            m1, m2, tpend, offv = carry
            tvec = tpend
            m1, m2, offv = filt8(half, it * (UPD * 16), m1, m2, tvec, offv, UPD)
            tpend = lax.broadcast_in_dim(jnp.min(m2), (16,), ())
            return m1, m2, tpend, offv

        carry = (neg16, neg16, neg16, jnp.zeros((16,), jnp.int32))
        m1, m2, tvec, offv = lax.fori_loop(0, NV // UPD, fbody, carry)
        m1, m2, offv = filt8(half, (NV - 1) * 16, m1, m2, tvec, offv, 1)
        nvs = jnp.max(offv) // 16
        tfin = lax.broadcast_in_dim(jnp.min(m2), (16,), ())
        for u in range(3):
            cval[pl.ds(nvs * 16 + 16 * u, 16)] = neg16
        off = compact((nvs + 3) // 4, tfin)

        hi_k, hi_i, lo_k, lo_i = _topk32_chain(cval, cidx, off)

        nv = (off + 15) // 16
        v32 = lax.broadcast_in_dim(jnp.min(lo_k), (16,), ())

        def cge(s, c):
            v = cval[pl.ds(s * 16, 16)]
            return c + plsc.all_reduce_population_count(v >= v32)

        cnt_ge = lax.fori_loop(0, nv, cge, jnp.zeros((16,), jnp.int32))
        sbv[pl.ds(0, 16)] = hi_k
        sbv[pl.ds(16, 16)] = lo_k
        e1 = plsc.load_gather(sbv, [iota]) == plsc.load_gather(sbv, [iota + 1])
        e2 = (plsc.load_gather(sbv, [iota + 16])
              == plsc.load_gather(sbv, [jnp.minimum(iota + 17, 31)]))
        eqc = plsc.all_reduce_population_count(e1 | (e2 & (iota < 15)))
        tied = (jnp.max(cnt_ge) != 32) | (jnp.max(eqc) > 0)

        hi_k, hi_i, lo_k, lo_i = lax.cond(
            tied,
            lambda: _topk32_exact(cval, cidx, off),
            lambda: (hi_k, hi_i, lo_k, lo_i),
        )

        sbv[pl.ds(0, 16)] = hi_k
        sbv[pl.ds(16, 16)] = lo_k
        sbi[pl.ds(0, 16)] = hi_i
        sbi[pl.ds(16, 16)] = lo_i
        q = s % 8
        qo = q * KOUT
        stg_v[pl.ds(qo, 16)] = plsc.load_gather(sbv, [sel16])
        stg_c[pl.ds(qo, 16)] = plsc.load_gather(sbi, [sel16])
        pos31 = lax.broadcast_in_dim(jnp.int32(31), (16,), ())
        last_pos = lax.broadcast_in_dim(qo + 16, (16,), ())
        plsc.store_scatter(stg_v, [last_pos], plsc.load_gather(sbv, [pos31]),
                           mask=lane0)
        plsc.store_scatter(stg_c, [last_pos], plsc.load_gather(sbi, [pos31]),
                           mask=lane0)
        row_id = row_of(s)
        stg_r[pl.ds(qo, 16)] = lax.broadcast_in_dim(row_id, (16,), ())
        plsc.store_scatter(stg_r, [last_pos],
                           lax.broadcast_in_dim(row_id, (16,), ()), mask=lane0)

    def sbody(s, carry):
        wait_fetch()

        @pl.when(s + 2 < ns)
        def _():
            fetch(s + 2, (s + 2) % 3)

        process_row(s)

        @pl.when(s % 8 == 7)
        def _():
            b = wid + NW * (s // 8)
            o = b * BLK_OUT
            pltpu.sync_copy(stg_r, rows_ref.at[pl.ds(o, BLK_OUT)])
            pltpu.sync_copy(stg_c, cols_ref.at[pl.ds(o, BLK_OUT)])
            pltpu.sync_copy(stg_v, vals_ref.at[pl.ds(o, BLK_OUT)])

        return carry

    lax.fori_loop(0, ns, sbody, jnp.int32(0))


def kernel(inputs):
    x1d = jnp.reshape(inputs, (-1,))
    mesh = plsc.VectorSubcoreMesh(
        core_axis_name="c", subcore_axis_name="s", num_cores=NC, num_subcores=NS
    )
    kern = pl.kernel(
        _body,
        out_type=(
            jax.ShapeDtypeStruct((N * KOUT,), jnp.int32),
            jax.ShapeDtypeStruct((N * KOUT,), jnp.int32),
            jax.ShapeDtypeStruct((N * KOUT,), jnp.float32),
        ),
        mesh=mesh,
        compiler_params=pltpu.CompilerParams(needs_layout_passes=False),
        scratch_types=[
            pltpu.VMEM((3 * N,), jnp.float32),
            pltpu.VMEM((CAPBUF,), jnp.float32),
            pltpu.VMEM((CAPBUF,), jnp.int32),
            pltpu.VMEM((TOPK,), jnp.float32),
            pltpu.VMEM((TOPK,), jnp.int32),
            pltpu.VMEM((BLK_OUT,), jnp.int32),
            pltpu.VMEM((BLK_OUT,), jnp.int32),
            pltpu.VMEM((BLK_OUT,), jnp.float32),
            pltpu.SemaphoreType.DMA,
        ],
    )
    rows, cols, values = kern(x1d)
    return rows, cols, values
